# baseline (speedup 1.0000x reference)
import functools
import math

import jax
import jax.numpy as jnp
from jax import lax
from jax.experimental import pallas as pl
from jax.experimental.pallas import tpu as pltpu
from jax.experimental.pallas import tpu_sc as plsc

S = 64.0
M = 0.5
T0 = 1.0
ALPHA = 0.01
B = 1024
C = 100000
COS_M = math.cos(M)
SIN_M = math.sin(M)
SHIFT = 2.0 * S


_NC = 2
_NS = 16
_L = 16
_NW = _NC * _NS
_B_PER_W = B // _NW


def _sc_gather_body(x_hbm, labels_hbm, out_hbm, lab_v, rows_v, vals_v, sem):
    wid = lax.axis_index("s") * _NC + lax.axis_index("c")
    base = wid * _B_PER_W
    pltpu.sync_copy(labels_hbm.at[pl.ds(base, _B_PER_W)], lab_v)
    iota16 = lax.iota(jnp.int32, _L)
    for j0 in range(0, _B_PER_W, _L):
        lab16 = lab_v[pl.ds(j0, _L)]
        cb16 = jnp.bitwise_and(lab16, -128)
        copies = []
        for k in range(_L):
            j = j0 + k
            row8 = base + (j // 8) * 8
            copies.append(
                pltpu.async_copy(
                    x_hbm.at[pl.ds(row8, 8), pl.ds(pl.multiple_of(cb16[k], 128), 128)],
                    rows_v.at[pl.ds(k * 8, 8), :],
                    sem,
                )
            )
        for cp in copies:
            cp.wait()
        lane16 = jnp.bitwise_and(lab16, 127)
        vals_res = jnp.zeros((_L,), jnp.float32)
        for k in range(_L):
            sub = (j0 + k) % 8
            lane_k = lane16[k]
            start_k = pl.multiple_of(jnp.bitwise_and(lane_k, -_L), _L)
            chunk = rows_v[k * 8 + sub, pl.ds(start_k, _L)]
            p_k = jnp.full((_L,), jnp.bitwise_and(lane_k, _L - 1), jnp.int32)
            v_vec = lax.gather(
                chunk,
                p_k[:, None],
                lax.GatherDimensionNumbers(
                    offset_dims=(), collapsed_slice_dims=(0,), start_index_map=(0,)
                ),
                slice_sizes=(1,),
                mode=lax.GatherScatterMode.PROMISE_IN_BOUNDS,
            )
            vals_res = jnp.where(iota16 == k, v_vec, vals_res)
        vals_v[pl.ds(j0, _L)] = vals_res
    pltpu.sync_copy(vals_v, out_hbm.at[pl.ds(base, _B_PER_W)])


@jax.jit
def _sc_gather(inputs, labels):
    fn = functools.partial(
        pl.kernel,
        mesh=plsc.VectorSubcoreMesh(core_axis_name="c", subcore_axis_name="s"),
        out_type=jax.ShapeDtypeStruct((B,), jnp.float32),
        scratch_types=[
            pltpu.VMEM((_B_PER_W,), jnp.int32),
            pltpu.VMEM((_L * 8, 128), jnp.float32),
            pltpu.VMEM((_B_PER_W,), jnp.float32),
            pltpu.SemaphoreType.DMA,
        ],
        compiler_params=pltpu.CompilerParams(use_tc_tiling_on_sc=True),
    )(_sc_gather_body)
    return fn(inputs, labels)


_BM = 128
_BN = 4096
_RB = B // _BM
_CB = (C + _BN - 1) // _BN


def _sweep_body(x_ref, tl_ref, out_ref, acc_ref, t_ref):
    i = pl.program_id(0)
    j = pl.program_id(1)

    @pl.when(jnp.logical_and(i == 0, j == 0))
    def _():
        tsum = jnp.sum(tl_ref[...])
        t_ref[0] = tsum * (ALPHA / B) + (1.0 - ALPHA) * T0
        out_ref[0, 0] = 0.0

    t_new = t_ref[0]
    tlb = tl_ref[pl.ds(i * _BM, _BM), :]
    ctm = tlb * COS_M - jnp.sqrt(1.0 - tlb * tlb) * SIN_M

    x = x_ref[...]
    hard = x > ctm
    mod = jnp.where(hard, x * (t_new + x), x)
    term = jnp.exp(mod * S - SHIFT)
    colid = j * _BN + lax.broadcasted_iota(jnp.int32, (_BM, _BN), 1)
    term = jnp.where(colid < C, term, 0.0)
    part = jnp.sum(term.reshape(_BM, _BN // 128, 128), axis=1)

    @pl.when(j == 0)
    def _():
        acc_ref[...] = part

    @pl.when(j > 0)
    def _():
        acc_ref[...] = acc_ref[...] + part

    @pl.when(j == _CB - 1)
    def _():
        row_sum = jnp.sum(acc_ref[...], axis=1, keepdims=True)
        lab_hard = tlb > ctm
        lab_mod = jnp.where(lab_hard, tlb * (t_new + tlb), tlb)
        lab_term = jnp.exp(lab_mod * S - SHIFT)
        ctm_term = jnp.exp(ctm * S - SHIFT)
        row_sum = row_sum - lab_term + ctm_term
        lse = SHIFT + jnp.log(row_sum)
        nll = lse - S * ctm
        out_ref[0, 0] += jnp.sum(nll) * (1.0 / B)


@jax.jit
def _tc_sweep(inputs, target_logit):
    tl2 = target_logit.reshape(B, 1)
    out = pl.pallas_call(
        _sweep_body,
        grid=(_RB, _CB),
        in_specs=[
            pl.BlockSpec((_BM, _BN), lambda i, j: (i, j)),
            pl.BlockSpec((B, 1), lambda i, j: (0, 0)),
        ],
        out_specs=pl.BlockSpec(memory_space=pltpu.SMEM),
        out_shape=jax.ShapeDtypeStruct((1, 1), jnp.float32),
        scratch_shapes=[
            pltpu.VMEM((_BM, 128), jnp.float32),
            pltpu.SMEM((1,), jnp.float32),
        ],
        compiler_params=pltpu.CompilerParams(
            dimension_semantics=("arbitrary", "arbitrary"),
        ),
    )(inputs, tl2)
    return out[0, 0]


def kernel(inputs, labels):
    target_logit = _sc_gather(inputs, labels)
    return _tc_sweep(inputs, target_logit)


# speedup vs baseline: 1.2640x; 1.2640x over previous
import functools
import math

import jax
import jax.numpy as jnp
from jax import lax
from jax.experimental import pallas as pl
from jax.experimental.pallas import tpu as pltpu
from jax.experimental.pallas import tpu_sc as plsc

S = 64.0
M = 0.5
T0 = 1.0
ALPHA = 0.01
B = 1024
C = 100000
COS_M = math.cos(M)
SIN_M = math.sin(M)
SHIFT = 2.0 * S


_NC = 2
_NS = 16
_L = 16
_NW = _NC * _NS
_B_PER_W = B // _NW


def _sc_gather_body(x_hbm, labels_hbm, out_hbm, lab_v, rows_v, vals_v, sem):
    wid = lax.axis_index("s") * _NC + lax.axis_index("c")
    base = wid * _B_PER_W
    pltpu.sync_copy(labels_hbm.at[pl.ds(base, _B_PER_W)], lab_v)
    iota16 = lax.iota(jnp.int32, _L)
    for j0 in range(0, _B_PER_W, _L):
        lab16 = lab_v[pl.ds(j0, _L)]
        cb16 = jnp.bitwise_and(lab16, -128)
        copies = []
        for k in range(_L):
            j = j0 + k
            row8 = base + (j // 8) * 8
            copies.append(
                pltpu.async_copy(
                    x_hbm.at[pl.ds(row8, 8), pl.ds(pl.multiple_of(cb16[k], 128), 128)],
                    rows_v.at[pl.ds(k * 8, 8), :],
                    sem,
                )
            )
        for cp in copies:
            cp.wait()
        lane16 = jnp.bitwise_and(lab16, 127)
        vals_res = jnp.zeros((_L,), jnp.float32)
        for k in range(_L):
            sub = (j0 + k) % 8
            lane_k = lane16[k]
            start_k = pl.multiple_of(jnp.bitwise_and(lane_k, -_L), _L)
            chunk = rows_v[k * 8 + sub, pl.ds(start_k, _L)]
            p_k = jnp.full((_L,), jnp.bitwise_and(lane_k, _L - 1), jnp.int32)
            v_vec = lax.gather(
                chunk,
                p_k[:, None],
                lax.GatherDimensionNumbers(
                    offset_dims=(), collapsed_slice_dims=(0,), start_index_map=(0,)
                ),
                slice_sizes=(1,),
                mode=lax.GatherScatterMode.PROMISE_IN_BOUNDS,
            )
            vals_res = jnp.where(iota16 == k, v_vec, vals_res)
        vals_v[pl.ds(j0, _L)] = vals_res
    pltpu.sync_copy(vals_v, out_hbm.at[pl.ds(base, _B_PER_W)])


@jax.jit
def _sc_gather(inputs, labels):
    fn = functools.partial(
        pl.kernel,
        mesh=plsc.VectorSubcoreMesh(core_axis_name="c", subcore_axis_name="s"),
        out_type=jax.ShapeDtypeStruct((B,), jnp.float32),
        scratch_types=[
            pltpu.VMEM((_B_PER_W,), jnp.int32),
            pltpu.VMEM((_L * 8, 128), jnp.float32),
            pltpu.VMEM((_B_PER_W,), jnp.float32),
            pltpu.SemaphoreType.DMA,
        ],
        compiler_params=pltpu.CompilerParams(use_tc_tiling_on_sc=True),
    )(_sc_gather_body)
    return fn(inputs, labels)


_BM = 8
_RB = B // _BM


def _sweep_body(x_ref, tl_ref, out_ref, t_ref):
    i = pl.program_id(0)

    @pl.when(i == 0)
    def _():
        tsum = jnp.sum(tl_ref[...])
        t_ref[0] = tsum * (ALPHA / B) + (1.0 - ALPHA) * T0
        out_ref[0, 0] = 0.0

    t_new = t_ref[0]
    tlb = tl_ref[pl.ds(i * _BM, _BM), :]
    ctm = tlb * COS_M - jnp.sqrt(1.0 - tlb * tlb) * SIN_M

    x = x_ref[...]
    hard = x > ctm
    mod = jnp.where(hard, x * (t_new + x), x)
    term = jnp.exp(mod * S - SHIFT)
    row_sum = jnp.sum(term, axis=1, keepdims=True)

    lab_hard = tlb > ctm
    lab_mod = jnp.where(lab_hard, tlb * (t_new + tlb), tlb)
    lab_term = jnp.exp(lab_mod * S - SHIFT)
    ctm_term = jnp.exp(ctm * S - SHIFT)
    row_sum = row_sum - lab_term + ctm_term
    lse = SHIFT + jnp.log(row_sum)
    nll = lse - S * ctm
    out_ref[0, 0] += jnp.sum(nll) * (1.0 / B)


@jax.jit
def _tc_sweep(inputs, target_logit):
    tl2 = target_logit.reshape(B, 1)
    out = pl.pallas_call(
        _sweep_body,
        grid=(_RB,),
        in_specs=[
            pl.BlockSpec((_BM, C), lambda i: (i, 0)),
            pl.BlockSpec((B, 1), lambda i: (0, 0)),
        ],
        out_specs=pl.BlockSpec(memory_space=pltpu.SMEM),
        out_shape=jax.ShapeDtypeStruct((1, 1), jnp.float32),
        scratch_shapes=[
            pltpu.SMEM((1,), jnp.float32),
        ],
        compiler_params=pltpu.CompilerParams(
            dimension_semantics=("arbitrary",),
        ),
    )(inputs, tl2)
    return out[0, 0]


def kernel(inputs, labels):
    target_logit = _sc_gather(inputs, labels)
    return _tc_sweep(inputs, target_logit)


# speedup vs baseline: 1.2700x; 1.0048x over previous
import functools
import math

import jax
import jax.numpy as jnp
from jax import lax
from jax.experimental import pallas as pl
from jax.experimental.pallas import tpu as pltpu
from jax.experimental.pallas import tpu_sc as plsc

S = 64.0
M = 0.5
T0 = 1.0
ALPHA = 0.01
B = 1024
C = 100000
COS_M = math.cos(M)
SIN_M = math.sin(M)
SHIFT = 2.0 * S


_NC = 2
_NS = 16
_L = 16
_NW = _NC * _NS
_B_PER_W = B // _NW


def _sc_gather_body(x_hbm, labels_hbm, out_hbm, lab_v, rows_v, vals_v, sem):
    wid = lax.axis_index("s") * _NC + lax.axis_index("c")
    base = wid * _B_PER_W
    pltpu.sync_copy(labels_hbm.at[pl.ds(base, _B_PER_W)], lab_v)
    iota16 = lax.iota(jnp.int32, _L)
    for j0 in range(0, _B_PER_W, _L):
        lab16 = lab_v[pl.ds(j0, _L)]
        cb16 = jnp.bitwise_and(lab16, -128)
        copies = []
        for k in range(_L):
            j = j0 + k
            row8 = base + (j // 8) * 8
            copies.append(
                pltpu.async_copy(
                    x_hbm.at[pl.ds(row8, 8), pl.ds(pl.multiple_of(cb16[k], 128), 128)],
                    rows_v.at[pl.ds(k * 8, 8), :],
                    sem,
                )
            )
        for cp in copies:
            cp.wait()
        lane16 = jnp.bitwise_and(lab16, 127)
        vals_res = jnp.zeros((_L,), jnp.float32)
        for k in range(_L):
            sub = (j0 + k) % 8
            lane_k = lane16[k]
            start_k = pl.multiple_of(jnp.bitwise_and(lane_k, -_L), _L)
            chunk = rows_v[k * 8 + sub, pl.ds(start_k, _L)]
            p_k = jnp.full((_L,), jnp.bitwise_and(lane_k, _L - 1), jnp.int32)
            v_vec = lax.gather(
                chunk,
                p_k[:, None],
                lax.GatherDimensionNumbers(
                    offset_dims=(), collapsed_slice_dims=(0,), start_index_map=(0,)
                ),
                slice_sizes=(1,),
                mode=lax.GatherScatterMode.PROMISE_IN_BOUNDS,
            )
            vals_res = jnp.where(iota16 == k, v_vec, vals_res)
        vals_v[pl.ds(j0, _L)] = vals_res
    pltpu.sync_copy(vals_v, out_hbm.at[pl.ds(base, _B_PER_W)])


@jax.jit
def _sc_gather(inputs, labels):
    fn = functools.partial(
        pl.kernel,
        mesh=plsc.VectorSubcoreMesh(core_axis_name="c", subcore_axis_name="s"),
        out_type=jax.ShapeDtypeStruct((B,), jnp.float32),
        scratch_types=[
            pltpu.VMEM((_B_PER_W,), jnp.int32),
            pltpu.VMEM((_L * 8, 128), jnp.float32),
            pltpu.VMEM((_B_PER_W,), jnp.float32),
            pltpu.SemaphoreType.DMA,
        ],
        compiler_params=pltpu.CompilerParams(use_tc_tiling_on_sc=True),
    )(_sc_gather_body)
    return fn(inputs, labels)


def _tc_gather_body(lab_smem, x_any, lab_v, out_ref, stage_ref, sem):
    def issue(r, c):
        cb = pl.multiple_of(jnp.bitwise_and(lab_smem[r], -128), 128)
        pltpu.make_async_copy(
            x_any.at[pl.ds(r, 1), pl.ds(cb, 128)],
            stage_ref.at[pl.ds(r, 1), :],
            sem,
        ).start()
        return c

    lax.fori_loop(0, B, issue, 0)

    def drain(r, c):
        pltpu.make_async_copy(
            x_any.at[pl.ds(0, 1), pl.ds(0, 128)],
            stage_ref.at[pl.ds(0, 1), :],
            sem,
        ).wait()
        return c

    lax.fori_loop(0, B, drain, 0)

    for b in range(B // 128):
        s = stage_ref[pl.ds(b * 128, 128), :]
        lane = jnp.bitwise_and(lab_v[pl.ds(b * 128, 128), :], 127)
        hit = lax.broadcasted_iota(jnp.int32, (128, 128), 1) == lane
        val = jnp.sum(jnp.where(hit, s, 0.0), axis=1, keepdims=True)
        out_ref[pl.ds(b * 128, 128), :] = val


@jax.jit
def _tc_gather(inputs, labels):
    out = pl.pallas_call(
        _tc_gather_body,
        in_specs=[
            pl.BlockSpec(memory_space=pltpu.SMEM),
            pl.BlockSpec(memory_space=pltpu.MemorySpace.HBM),
            pl.BlockSpec((B, 1), lambda: (0, 0)),
        ],
        out_specs=pl.BlockSpec((B, 1), lambda: (0, 0)),
        out_shape=jax.ShapeDtypeStruct((B, 1), jnp.float32),
        scratch_shapes=[
            pltpu.VMEM((B, 128), jnp.float32),
            pltpu.SemaphoreType.DMA,
        ],
    )(labels, inputs, labels.reshape(B, 1))
    return out


_BM = 8
_RB = B // _BM


def _sweep_body(x_ref, tl_ref, out_ref, t_ref):
    i = pl.program_id(0)

    @pl.when(i == 0)
    def _():
        tsum = jnp.sum(tl_ref[...])
        t_ref[0] = tsum * (ALPHA / B) + (1.0 - ALPHA) * T0
        out_ref[0, 0] = 0.0

    t_new = t_ref[0]
    tlb = tl_ref[pl.ds(i * _BM, _BM), :]
    ctm = tlb * COS_M - jnp.sqrt(1.0 - tlb * tlb) * SIN_M

    x = x_ref[...]
    hard = x > ctm
    mod = jnp.where(hard, x * (t_new + x), x)
    term = jnp.exp(mod * S - SHIFT)
    row_sum = jnp.sum(term, axis=1, keepdims=True)

    lab_hard = tlb > ctm
    lab_mod = jnp.where(lab_hard, tlb * (t_new + tlb), tlb)
    lab_term = jnp.exp(lab_mod * S - SHIFT)
    ctm_term = jnp.exp(ctm * S - SHIFT)
    row_sum = row_sum - lab_term + ctm_term
    lse = SHIFT + jnp.log(row_sum)
    nll = lse - S * ctm
    out_ref[0, 0] += jnp.sum(nll) * (1.0 / B)


@jax.jit
def _tc_sweep(inputs, target_logit):
    tl2 = target_logit.reshape(B, 1)
    out = pl.pallas_call(
        _sweep_body,
        grid=(_RB,),
        in_specs=[
            pl.BlockSpec((_BM, C), lambda i: (i, 0)),
            pl.BlockSpec((B, 1), lambda i: (0, 0)),
        ],
        out_specs=pl.BlockSpec(memory_space=pltpu.SMEM),
        out_shape=jax.ShapeDtypeStruct((1, 1), jnp.float32),
        scratch_shapes=[
            pltpu.SMEM((1,), jnp.float32),
        ],
        compiler_params=pltpu.CompilerParams(
            dimension_semantics=("arbitrary",),
        ),
    )(inputs, tl2)
    return out[0, 0]


def kernel(inputs, labels):
    target_logit = _tc_gather(inputs, labels)
    return _tc_sweep(inputs, target_logit)


# speedup vs baseline: 3.0991x; 2.4402x over previous
import functools
import math

import jax
import jax.numpy as jnp
from jax import lax
from jax.experimental import pallas as pl
from jax.experimental.pallas import tpu as pltpu

S = 64.0
M = 0.5
T0 = 1.0
ALPHA = 0.01
B = 1024
C = 100000
COS_M = math.cos(M)
SIN_M = math.sin(M)
SHIFT = 2.0 * S


from jax.experimental.pallas import tpu_sc as plsc

_NC = 2
_NS = 16
_L = 16
_NW = _NC * _NS
_B_PER_W = B // _NW


def _sc_gather_body(x_hbm, labels_hbm, out_hbm, lab_v, rows_v, vals_v, sem):
    wid = lax.axis_index("s") * _NC + lax.axis_index("c")
    base = wid * _B_PER_W
    pltpu.sync_copy(labels_hbm.at[pl.ds(base, _B_PER_W)], lab_v)
    iota16 = lax.iota(jnp.int32, _L)
    for j0 in range(0, _B_PER_W, _L):
        lab16 = lab_v[pl.ds(j0, _L)]
        cb16 = jnp.bitwise_and(lab16, -8)
        lane_start = pl.multiple_of(jnp.bitwise_and(base + j0, -128), 128)
        copies = []
        for k in range(_L):
            copies.append(
                pltpu.async_copy(
                    x_hbm.at[
                        pl.ds(pl.multiple_of(cb16[k], 8), 8),
                        pl.ds(lane_start, 128),
                    ],
                    rows_v.at[pl.ds(k * 8, 8), :],
                    sem,
                )
            )
        for cp in copies:
            cp.wait()
        lw = pl.multiple_of(jnp.bitwise_and(base + j0, 127), _L)
        sub16 = jnp.bitwise_and(lab16, 7)
        res = jnp.zeros((_L,), jnp.float32)
        for k in range(_L):
            svec = jnp.full((_L,), sub16[k], jnp.int32)
            onehot_k = (1 - jnp.minimum(jnp.abs(iota16 - k), 1)).astype(jnp.float32)
            sel = jnp.zeros((_L,), jnp.float32)
            for s in range(8):
                chunk = rows_v[k * 8 + s, pl.ds(lw, _L)]
                ind = (1 - jnp.minimum(jnp.abs(svec - s), 1)).astype(jnp.float32)
                sel = sel + chunk * ind
            res = res + sel * onehot_k
        vals_v[pl.ds(j0, _L)] = res
    pltpu.sync_copy(vals_v, out_hbm.at[pl.ds(base, _B_PER_W)])


@jax.jit
def _sc_gather(x_t, labels):
    fn = functools.partial(
        pl.kernel,
        mesh=plsc.VectorSubcoreMesh(core_axis_name="c", subcore_axis_name="s"),
        out_type=jax.ShapeDtypeStruct((B,), jnp.float32),
        scratch_types=[
            pltpu.VMEM((_B_PER_W,), jnp.int32),
            pltpu.VMEM((_L * 8, 128), jnp.float32),
            pltpu.VMEM((_B_PER_W,), jnp.float32),
            pltpu.SemaphoreType.DMA,
        ],
        compiler_params=pltpu.CompilerParams(use_tc_tiling_on_sc=True),
    )(_sc_gather_body)
    return fn(x_t, labels)


_BR = 2000
_NSTEP = C // _BR


def _sweep_body(x_ref, tl_ref, out_ref, acc_ref, t_ref):
    i = pl.program_id(0)

    tlb = tl_ref[...]

    @pl.when(i == 0)
    def _():
        t_ref[0] = jnp.sum(tlb) * (ALPHA / B) + (1.0 - ALPHA) * T0
        acc_ref[...] = jnp.zeros((1, B), jnp.float32)

    t_new = t_ref[0]
    ctm = tlb * COS_M - jnp.sqrt(1.0 - tlb * tlb) * SIN_M

    x = x_ref[...]
    hard = x > ctm
    mod = jnp.where(hard, x * (t_new + x), x)
    term = jnp.exp(mod * S - SHIFT)
    acc_ref[...] += jnp.sum(term, axis=0, keepdims=True)

    @pl.when(i == _NSTEP - 1)
    def _():
        lab_hard = tlb > ctm
        lab_mod = jnp.where(lab_hard, tlb * (t_new + tlb), tlb)
        lab_term = jnp.exp(lab_mod * S - SHIFT)
        ctm_term = jnp.exp(ctm * S - SHIFT)
        col_sum = acc_ref[...] - lab_term + ctm_term
        lse = SHIFT + jnp.log(col_sum)
        nll = lse - S * ctm
        out_ref[0, 0] = jnp.sum(nll) * (1.0 / B)


@jax.jit
def _tc_sweep(x_t, target_logit):
    out = pl.pallas_call(
        _sweep_body,
        grid=(_NSTEP,),
        in_specs=[
            pl.BlockSpec((_BR, B), lambda i: (i, 0)),
            pl.BlockSpec((1, B), lambda i: (0, 0)),
        ],
        out_specs=pl.BlockSpec(memory_space=pltpu.SMEM),
        out_shape=jax.ShapeDtypeStruct((1, 1), jnp.float32),
        scratch_shapes=[
            pltpu.VMEM((1, B), jnp.float32),
            pltpu.SMEM((1,), jnp.float32),
        ],
        compiler_params=pltpu.CompilerParams(
            dimension_semantics=("arbitrary",),
        ),
    )(x_t, target_logit)
    return out[0, 0]


def kernel(inputs, labels):
    x_t = inputs.T
    target_logit = _sc_gather(x_t, labels)
    return _tc_sweep(x_t, target_logit.reshape(1, B))


# speedup vs baseline: 3.1438x; 1.0144x over previous
import functools
import math

import jax
import jax.numpy as jnp
from jax import lax
from jax.experimental import pallas as pl
from jax.experimental.pallas import tpu as pltpu

S = 64.0
M = 0.5
T0 = 1.0
ALPHA = 0.01
B = 1024
C = 100000
COS_M = math.cos(M)
SIN_M = math.sin(M)
SHIFT = 2.0 * S


from jax.experimental.pallas import tpu_sc as plsc

_NC = 2
_NS = 16
_L = 16
_NW = _NC * _NS
_B_PER_W = B // _NW


def _sc_gather_body(x_hbm, labels_hbm, out_hbm, lab_v, rows_v, vals_v, sem):
    wid = lax.axis_index("s") * _NC + lax.axis_index("c")
    base = wid * _B_PER_W
    pltpu.sync_copy(labels_hbm.at[pl.ds(base, _B_PER_W)], lab_v)
    iota16 = lax.iota(jnp.int32, _L)
    for j0 in range(0, _B_PER_W, _L):
        lab16 = lab_v[pl.ds(j0, _L)]
        cb16 = jnp.bitwise_and(lab16, -8)
        lane_start = pl.multiple_of(jnp.bitwise_and(base + j0, -128), 128)
        copies = []
        for k in range(_L):
            copies.append(
                pltpu.async_copy(
                    x_hbm.at[
                        pl.ds(pl.multiple_of(cb16[k], 8), 8),
                        pl.ds(lane_start, 128),
                    ],
                    rows_v.at[pl.ds(k * 8, 8), :],
                    sem,
                )
            )
        for cp in copies:
            cp.wait()
        lw = pl.multiple_of(jnp.bitwise_and(base + j0, 127), _L)
        sub16 = jnp.bitwise_and(lab16, 7)
        res = jnp.zeros((_L,), jnp.float32)
        for k in range(_L):
            svec = jnp.full((_L,), sub16[k], jnp.int32)
            onehot_k = (1 - jnp.minimum(jnp.abs(iota16 - k), 1)).astype(jnp.float32)
            sel = jnp.zeros((_L,), jnp.float32)
            for s in range(8):
                chunk = rows_v[k * 8 + s, pl.ds(lw, _L)]
                ind = (1 - jnp.minimum(jnp.abs(svec - s), 1)).astype(jnp.float32)
                sel = sel + chunk * ind
            res = res + sel * onehot_k
        vals_v[pl.ds(j0, _L)] = res
    pltpu.sync_copy(vals_v, out_hbm.at[pl.ds(base, _B_PER_W)])


@jax.jit
def _sc_gather(x_t, labels):
    fn = functools.partial(
        pl.kernel,
        mesh=plsc.VectorSubcoreMesh(core_axis_name="c", subcore_axis_name="s"),
        out_type=jax.ShapeDtypeStruct((B,), jnp.float32),
        scratch_types=[
            pltpu.VMEM((_B_PER_W,), jnp.int32),
            pltpu.VMEM((_L * 8, 128), jnp.float32),
            pltpu.VMEM((_B_PER_W,), jnp.float32),
            pltpu.SemaphoreType.DMA,
        ],
        compiler_params=pltpu.CompilerParams(use_tc_tiling_on_sc=True),
    )(_sc_gather_body)
    return fn(x_t, labels)


_BR = 4000
_NSTEP = C // _BR
_LOG2E = 1.4426950408889634
_K1 = S * _LOG2E
_K2 = SHIFT * _LOG2E


def _sweep_body(x_ref, tl_ref, out_ref, acc_ref, ctm_ref, t_ref):
    i = pl.program_id(0)

    @pl.when(i == 0)
    def _():
        tlb = tl_ref[...]
        t_ref[0] = jnp.sum(tlb) * (ALPHA / B) + (1.0 - ALPHA) * T0
        ctm_ref[...] = tlb * COS_M - jnp.sqrt(1.0 - tlb * tlb) * SIN_M
        acc_ref[...] = jnp.zeros((1, B), jnp.float32)

    t_new = t_ref[0]
    ctm = ctm_ref[...]

    x = x_ref[...]
    hard = x > ctm
    mod = jnp.where(hard, x * (t_new + x), x)
    term = jnp.exp2(mod * _K1 - _K2)
    acc_ref[...] += jnp.sum(term, axis=0, keepdims=True)

    @pl.when(i == _NSTEP - 1)
    def _():
        tlb = tl_ref[...]
        lab_hard = tlb > ctm
        lab_mod = jnp.where(lab_hard, tlb * (t_new + tlb), tlb)
        lab_term = jnp.exp2(lab_mod * _K1 - _K2)
        ctm_term = jnp.exp2(ctm * _K1 - _K2)
        col_sum = acc_ref[...] - lab_term + ctm_term
        lse = SHIFT + jnp.log(col_sum)
        nll = lse - S * ctm
        out_ref[0, 0] = jnp.sum(nll) * (1.0 / B)


@jax.jit
def _tc_sweep(x_t, target_logit):
    out = pl.pallas_call(
        _sweep_body,
        grid=(_NSTEP,),
        in_specs=[
            pl.BlockSpec((_BR, B), lambda i: (i, 0)),
            pl.BlockSpec((1, B), lambda i: (0, 0)),
        ],
        out_specs=pl.BlockSpec(memory_space=pltpu.SMEM),
        out_shape=jax.ShapeDtypeStruct((1, 1), jnp.float32),
        scratch_shapes=[
            pltpu.VMEM((1, B), jnp.float32),
            pltpu.VMEM((1, B), jnp.float32),
            pltpu.SMEM((1,), jnp.float32),
        ],
        compiler_params=pltpu.CompilerParams(
            dimension_semantics=("arbitrary",),
        ),
    )(x_t, target_logit)
    return out[0, 0]


def kernel(inputs, labels):
    x_t = inputs.T
    target_logit = _sc_gather(x_t, labels)
    return _tc_sweep(x_t, target_logit.reshape(1, B))


# speedup vs baseline: 4.7085x; 1.4977x over previous
import functools
import math

import jax
import jax.numpy as jnp
from jax import lax
from jax.experimental import pallas as pl
from jax.experimental.pallas import tpu as pltpu

S = 64.0
M = 0.5
T0 = 1.0
ALPHA = 0.01
B = 1024
C = 100000
COS_M = math.cos(M)
SIN_M = math.sin(M)
SHIFT = 2.0 * S


from jax.experimental.pallas import tpu_sc as plsc

_NC = 2
_NS = 16
_L = 16
_NW = _NC * _NS
_B_PER_W = B // _NW


def _sc_gather_body(x_hbm, labels_hbm, out_hbm, lab_v, rows_v, vals_v, sem):
    wid = lax.axis_index("s") * _NC + lax.axis_index("c")
    base = wid * _B_PER_W
    pltpu.sync_copy(labels_hbm.at[pl.ds(base, _B_PER_W)], lab_v)
    iota16 = lax.iota(jnp.int32, _L)
    for j0 in range(0, _B_PER_W, _L):
        lab16 = lab_v[pl.ds(j0, _L)]
        cb16 = jnp.bitwise_and(lab16, -8)
        lane_start = pl.multiple_of(jnp.bitwise_and(base + j0, -128), 128)
        copies = []
        for k in range(_L):
            copies.append(
                pltpu.async_copy(
                    x_hbm.at[
                        pl.ds(pl.multiple_of(cb16[k], 8), 8),
                        pl.ds(lane_start, 128),
                    ],
                    rows_v.at[pl.ds(k * 8, 8), :],
                    sem,
                )
            )
        for cp in copies:
            cp.wait()
        lw = pl.multiple_of(jnp.bitwise_and(base + j0, 127), _L)
        sub16 = jnp.bitwise_and(lab16, 7)
        res = jnp.zeros((_L,), jnp.float32)
        for k in range(_L):
            svec = jnp.full((_L,), sub16[k], jnp.int32)
            onehot_k = (1 - jnp.minimum(jnp.abs(iota16 - k), 1)).astype(jnp.float32)
            sel = jnp.zeros((_L,), jnp.float32)
            for s in range(8):
                chunk = rows_v[k * 8 + s, pl.ds(lw, _L)]
                ind = (1 - jnp.minimum(jnp.abs(svec - s), 1)).astype(jnp.float32)
                sel = sel + chunk * ind
            res = res + sel * onehot_k
        vals_v[pl.ds(j0, _L)] = res
    pltpu.sync_copy(vals_v, out_hbm.at[pl.ds(base, _B_PER_W)])


@jax.jit
def _sc_gather(x_t, labels):
    fn = functools.partial(
        pl.kernel,
        mesh=plsc.VectorSubcoreMesh(core_axis_name="c", subcore_axis_name="s"),
        out_type=jax.ShapeDtypeStruct((B,), jnp.float32),
        scratch_types=[
            pltpu.VMEM((_B_PER_W,), jnp.int32),
            pltpu.VMEM((_L * 8, 128), jnp.float32),
            pltpu.VMEM((_B_PER_W,), jnp.float32),
            pltpu.SemaphoreType.DMA,
        ],
        compiler_params=pltpu.CompilerParams(use_tc_tiling_on_sc=True),
    )(_sc_gather_body)
    return fn(x_t, labels)


_BR = 4000
_NSTEP = C // _BR
_LOG2E = 1.4426950408889634
_K1 = S * _LOG2E
_K2 = SHIFT * _LOG2E


def _sweep_body(x_ref, tl_ref, out_ref, acc_ref, ctm_ref, t_ref):
    i = pl.program_id(0)

    @pl.when(i == 0)
    def _():
        tlb = tl_ref[...]
        t_ref[0] = jnp.sum(tlb) * (ALPHA / B) + (1.0 - ALPHA) * T0
        ctm_ref[...] = tlb * COS_M - jnp.sqrt(1.0 - tlb * tlb) * SIN_M
        acc_ref[...] = jnp.zeros((8, B), jnp.float32)

    t_new = t_ref[0]
    ctm = ctm_ref[...]

    acc8 = acc_ref[...]
    for g in range(_BR // 32):
        xg = x_ref[pl.ds(g * 32, 32), :]
        hard = xg > ctm
        mod = jnp.where(hard, xg * (t_new + xg), xg)
        term = jnp.exp2(mod * _K1 - _K2)
        acc8 = acc8 + (
            (term[0:8] + term[8:16]) + (term[16:24] + term[24:32])
        )
    acc_ref[...] = acc8

    @pl.when(i == _NSTEP - 1)
    def _():
        tlb = tl_ref[...]
        lab_hard = tlb > ctm
        lab_mod = jnp.where(lab_hard, tlb * (t_new + tlb), tlb)
        lab_term = jnp.exp2(lab_mod * _K1 - _K2)
        ctm_term = jnp.exp2(ctm * _K1 - _K2)
        col_sum = jnp.sum(acc8, axis=0, keepdims=True) - lab_term + ctm_term
        lse = SHIFT + jnp.log(col_sum)
        nll = lse - S * ctm
        out_ref[0, 0] = jnp.sum(nll) * (1.0 / B)


@jax.jit
def _tc_sweep(x_t, target_logit):
    out = pl.pallas_call(
        _sweep_body,
        grid=(_NSTEP,),
        in_specs=[
            pl.BlockSpec((_BR, B), lambda i: (i, 0)),
            pl.BlockSpec((1, B), lambda i: (0, 0)),
        ],
        out_specs=pl.BlockSpec(memory_space=pltpu.SMEM),
        out_shape=jax.ShapeDtypeStruct((1, 1), jnp.float32),
        scratch_shapes=[
            pltpu.VMEM((8, B), jnp.float32),
            pltpu.VMEM((1, B), jnp.float32),
            pltpu.SMEM((1,), jnp.float32),
        ],
        compiler_params=pltpu.CompilerParams(
            dimension_semantics=("arbitrary",),
        ),
    )(x_t, target_logit)
    return out[0, 0]


def kernel(inputs, labels):
    x_t = inputs.T
    target_logit = _sc_gather(x_t, labels)
    return _tc_sweep(x_t, target_logit.reshape(1, B))


# speedup vs baseline: 4.7098x; 1.0003x over previous
import functools
import math

import jax
import jax.numpy as jnp
from jax import lax
from jax.experimental import pallas as pl
from jax.experimental.pallas import tpu as pltpu

S = 64.0
M = 0.5
T0 = 1.0
ALPHA = 0.01
B = 1024
C = 100000
COS_M = math.cos(M)
SIN_M = math.sin(M)
SHIFT = 2.0 * S


from jax.experimental.pallas import tpu_sc as plsc

_NC = 2
_NS = 16
_L = 16
_NW = _NC * _NS
_B_PER_W = B // _NW


def _sc_gather_body(x_hbm, labels_hbm, out_hbm, lab_v, rows_v, vals_v, sem):
    wid = lax.axis_index("s") * _NC + lax.axis_index("c")
    base = wid * _B_PER_W
    pltpu.sync_copy(labels_hbm.at[pl.ds(base, _B_PER_W)], lab_v)
    iota16 = lax.iota(jnp.int32, _L)
    for j0 in range(0, _B_PER_W, _L):
        lab16 = lab_v[pl.ds(j0, _L)]
        cb16 = jnp.bitwise_and(lab16, -8)
        lane_start = pl.multiple_of(jnp.bitwise_and(base + j0, -128), 128)
        copies = []
        for k in range(_L):
            copies.append(
                pltpu.async_copy(
                    x_hbm.at[
                        pl.ds(pl.multiple_of(cb16[k], 8), 8),
                        pl.ds(lane_start, 128),
                    ],
                    rows_v.at[pl.ds(k * 8, 8), :],
                    sem,
                )
            )
        for cp in copies:
            cp.wait()
        lw = pl.multiple_of(jnp.bitwise_and(base + j0, 127), _L)
        sub16 = jnp.bitwise_and(lab16, 7)
        res = jnp.zeros((_L,), jnp.float32)
        for k in range(_L):
            svec = jnp.full((_L,), sub16[k], jnp.int32)
            onehot_k = (1 - jnp.minimum(jnp.abs(iota16 - k), 1)).astype(jnp.float32)
            sel = jnp.zeros((_L,), jnp.float32)
            for s in range(8):
                chunk = rows_v[k * 8 + s, pl.ds(lw, _L)]
                ind = (1 - jnp.minimum(jnp.abs(svec - s), 1)).astype(jnp.float32)
                sel = sel + chunk * ind
            res = res + sel * onehot_k
        vals_v[pl.ds(j0, _L)] = res
    pltpu.sync_copy(vals_v, out_hbm.at[pl.ds(base, _B_PER_W)])


@jax.jit
def _sc_gather(x_t, labels):
    fn = functools.partial(
        pl.kernel,
        mesh=plsc.VectorSubcoreMesh(core_axis_name="c", subcore_axis_name="s"),
        out_type=jax.ShapeDtypeStruct((B,), jnp.float32),
        scratch_types=[
            pltpu.VMEM((_B_PER_W,), jnp.int32),
            pltpu.VMEM((_L * 8, 128), jnp.float32),
            pltpu.VMEM((_B_PER_W,), jnp.float32),
            pltpu.SemaphoreType.DMA,
        ],
        compiler_params=pltpu.CompilerParams(use_tc_tiling_on_sc=True),
    )(_sc_gather_body)
    return fn(x_t, labels)


_BR = 2000
_NSTEP = C // (2 * _BR)
_LOG2E = 1.4426950408889634
_K1 = S * _LOG2E
_K2 = SHIFT * _LOG2E


def _sweep_body(xa_ref, xb_ref, tl_ref, out_ref, acc_ref, ctm_ref, t_ref):
    i = pl.program_id(0)

    @pl.when(i == 0)
    def _():
        tlb = tl_ref[...]
        t_ref[0] = jnp.sum(tlb) * (ALPHA / B) + (1.0 - ALPHA) * T0
        ctm_ref[...] = tlb * COS_M - jnp.sqrt(1.0 - tlb * tlb) * SIN_M
        acc_ref[...] = jnp.zeros((8, B), jnp.float32)

    t_new = t_ref[0]
    ctm = ctm_ref[...]

    acc8 = acc_ref[...]
    for ref in (xa_ref, xb_ref):
        for g in range(_BR // 16):
            xg = ref[pl.ds(g * 16, 16), :]
            hard = xg > ctm
            mod = jnp.where(hard, xg * (t_new + xg), xg)
            term = jnp.exp2(mod * _K1 - _K2)
            acc8 = acc8 + (term[0:8] + term[8:16])
    acc_ref[...] = acc8

    @pl.when(i == _NSTEP - 1)
    def _():
        tlb = tl_ref[...]
        lab_hard = tlb > ctm
        lab_mod = jnp.where(lab_hard, tlb * (t_new + tlb), tlb)
        lab_term = jnp.exp2(lab_mod * _K1 - _K2)
        ctm_term = jnp.exp2(ctm * _K1 - _K2)
        col_sum = jnp.sum(acc8, axis=0, keepdims=True) - lab_term + ctm_term
        lse = SHIFT + jnp.log(col_sum)
        nll = lse - S * ctm
        out_ref[0, 0] = jnp.sum(nll) * (1.0 / B)


@jax.jit
def _tc_sweep(x_t, target_logit):
    out = pl.pallas_call(
        _sweep_body,
        grid=(_NSTEP,),
        in_specs=[
            pl.BlockSpec((_BR, B), lambda i: (2 * i, 0)),
            pl.BlockSpec((_BR, B), lambda i: (2 * i + 1, 0)),
            pl.BlockSpec((1, B), lambda i: (0, 0)),
        ],
        out_specs=pl.BlockSpec(memory_space=pltpu.SMEM),
        out_shape=jax.ShapeDtypeStruct((1, 1), jnp.float32),
        scratch_shapes=[
            pltpu.VMEM((8, B), jnp.float32),
            pltpu.VMEM((1, B), jnp.float32),
            pltpu.SMEM((1,), jnp.float32),
        ],
        compiler_params=pltpu.CompilerParams(
            dimension_semantics=("arbitrary",),
        ),
    )(x_t, x_t, target_logit)
    return out[0, 0]


def kernel(inputs, labels):
    x_t = inputs.T
    target_logit = _sc_gather(x_t, labels)
    return _tc_sweep(x_t, target_logit.reshape(1, B))


# speedup vs baseline: 4.7204x; 1.0023x over previous
import functools
import math

import jax
import jax.numpy as jnp
from jax import lax
from jax.experimental import pallas as pl
from jax.experimental.pallas import tpu as pltpu

S = 64.0
M = 0.5
T0 = 1.0
ALPHA = 0.01
B = 1024
C = 100000
COS_M = math.cos(M)
SIN_M = math.sin(M)
SHIFT = 2.0 * S


from jax.experimental.pallas import tpu_sc as plsc

_NC = 2
_NS = 16
_L = 16
_NW = _NC * _NS
_B_PER_W = B // _NW


def _sc_gather_body(x_hbm, labels_hbm, out_hbm, lab_v, rows_v, vals_v, sem):
    wid = lax.axis_index("s") * _NC + lax.axis_index("c")
    base = wid * _B_PER_W
    pltpu.sync_copy(labels_hbm.at[pl.ds(base, _B_PER_W)], lab_v)
    iota16 = lax.iota(jnp.int32, _L)
    copies = []
    for j0 in range(0, _B_PER_W, _L):
        lab16 = lab_v[pl.ds(j0, _L)]
        cb16 = jnp.bitwise_and(lab16, -8)
        lane_start = pl.multiple_of(jnp.bitwise_and(base + j0, -128), 128)
        for k in range(_L):
            copies.append(
                pltpu.async_copy(
                    x_hbm.at[
                        pl.ds(pl.multiple_of(cb16[k], 8), 8),
                        pl.ds(lane_start, 128),
                    ],
                    rows_v.at[pl.ds((j0 + k) * 8, 8), :],
                    sem,
                )
            )
    for cp in copies:
        cp.wait()
    for j0 in range(0, _B_PER_W, _L):
        lab16 = lab_v[pl.ds(j0, _L)]
        lw = pl.multiple_of(jnp.bitwise_and(base + j0, 127), _L)
        sub16 = jnp.bitwise_and(lab16, 7)
        res = jnp.zeros((_L,), jnp.float32)
        for k in range(_L):
            svec = jnp.full((_L,), sub16[k], jnp.int32)
            onehot_k = (1 - jnp.minimum(jnp.abs(iota16 - k), 1)).astype(jnp.float32)
            sel = jnp.zeros((_L,), jnp.float32)
            for s in range(8):
                chunk = rows_v[(j0 + k) * 8 + s, pl.ds(lw, _L)]
                ind = (1 - jnp.minimum(jnp.abs(svec - s), 1)).astype(jnp.float32)
                sel = sel + chunk * ind
            res = res + sel * onehot_k
        vals_v[pl.ds(j0, _L)] = res
    pltpu.sync_copy(vals_v, out_hbm.at[pl.ds(base, _B_PER_W)])


@jax.jit
def _sc_gather(x_t, labels):
    fn = functools.partial(
        pl.kernel,
        mesh=plsc.VectorSubcoreMesh(core_axis_name="c", subcore_axis_name="s"),
        out_type=jax.ShapeDtypeStruct((B,), jnp.float32),
        scratch_types=[
            pltpu.VMEM((_B_PER_W,), jnp.int32),
            pltpu.VMEM((_B_PER_W * 8, 128), jnp.float32),
            pltpu.VMEM((_B_PER_W,), jnp.float32),
            pltpu.SemaphoreType.DMA,
        ],
        compiler_params=pltpu.CompilerParams(use_tc_tiling_on_sc=True),
    )(_sc_gather_body)
    return fn(x_t, labels)


_BR = 2000
_NSTEP = C // (2 * _BR)
_LOG2E = 1.4426950408889634
_K1 = S * _LOG2E
_K2 = SHIFT * _LOG2E


def _sweep_body(xa_ref, xb_ref, tl_ref, out_ref, acc_ref, ctm_ref, t_ref):
    i = pl.program_id(0)

    @pl.when(i == 0)
    def _():
        tlb = tl_ref[...]
        t_ref[0] = jnp.sum(tlb) * (ALPHA / B) + (1.0 - ALPHA) * T0
        ctm_ref[...] = tlb * COS_M - jnp.sqrt(1.0 - tlb * tlb) * SIN_M
        acc_ref[...] = jnp.zeros((8, B), jnp.float32)

    t_new = t_ref[0]
    ctm = ctm_ref[...]

    acc8 = acc_ref[...]
    for ref in (xa_ref, xb_ref):
        for g in range(_BR // 16):
            xg = ref[pl.ds(g * 16, 16), :]
            hard = xg > ctm
            mod = jnp.where(hard, xg * (t_new + xg), xg)
            term = jnp.exp2(mod * _K1 - _K2)
            acc8 = acc8 + (term[0:8] + term[8:16])
    acc_ref[...] = acc8

    @pl.when(i == _NSTEP - 1)
    def _():
        tlb = tl_ref[...]
        lab_hard = tlb > ctm
        lab_mod = jnp.where(lab_hard, tlb * (t_new + tlb), tlb)
        lab_term = jnp.exp2(lab_mod * _K1 - _K2)
        ctm_term = jnp.exp2(ctm * _K1 - _K2)
        col_sum = jnp.sum(acc8, axis=0, keepdims=True) - lab_term + ctm_term
        lse = SHIFT + jnp.log(col_sum)
        nll = lse - S * ctm
        out_ref[0, 0] = jnp.sum(nll) * (1.0 / B)


@jax.jit
def _tc_sweep(x_t, target_logit):
    out = pl.pallas_call(
        _sweep_body,
        grid=(_NSTEP,),
        in_specs=[
            pl.BlockSpec((_BR, B), lambda i: (2 * i, 0)),
            pl.BlockSpec((_BR, B), lambda i: (2 * i + 1, 0)),
            pl.BlockSpec((1, B), lambda i: (0, 0)),
        ],
        out_specs=pl.BlockSpec(memory_space=pltpu.SMEM),
        out_shape=jax.ShapeDtypeStruct((1, 1), jnp.float32),
        scratch_shapes=[
            pltpu.VMEM((8, B), jnp.float32),
            pltpu.VMEM((1, B), jnp.float32),
            pltpu.SMEM((1,), jnp.float32),
        ],
        compiler_params=pltpu.CompilerParams(
            dimension_semantics=("arbitrary",),
        ),
    )(x_t, x_t, target_logit)
    return out[0, 0]


def kernel(inputs, labels):
    x_t = inputs.T
    target_logit = _sc_gather(x_t, labels)
    return _tc_sweep(x_t, target_logit.reshape(1, B))
